# baseline (speedup 1.0000x reference)
import functools

import jax
import jax.numpy as jnp
from jax import lax
from jax.experimental import pallas as pl
from jax.experimental.pallas import tpu as pltpu
from jax.experimental.pallas import tpu_sc as plsc

_LANES = 128
_K = 10
_NBUF = 4


def _emb_lookup(weight_bf16, idx_rows):
    R = idx_rows.shape[0]
    _, D = weight_bf16.shape
    info = plsc.get_sparse_core_info()
    num_cores = info.num_cores
    nw = num_cores * info.num_subcores
    rows_per_w = R // nw
    groups = rows_per_w // _K
    assert groups % _NBUF == 0
    rounds = groups // _NBUF
    gsz = _K * _LANES

    mesh = plsc.VectorSubcoreMesh(core_axis_name="c", subcore_axis_name="s")

    @functools.partial(
        pl.kernel,
        mesh=mesh,
        compiler_params=pltpu.CompilerParams(use_tc_tiling_on_sc=False),
        out_type=jax.ShapeDtypeStruct((R * _LANES, D), jnp.bfloat16),
        scratch_types=[
            [pltpu.VMEM((_K, _LANES), jnp.int32) for _ in range(_NBUF)],
            [pltpu.VMEM((gsz, D), jnp.bfloat16) for _ in range(_NBUF)],
            [pltpu.SemaphoreType.DMA for _ in range(_NBUF)],
            [pltpu.SemaphoreType.DMA for _ in range(_NBUF)],
        ],
    )
    def emb(w_hbm, idx_hbm, out_hbm, idx_v, rows_v, gsem, ssem):
        wid = lax.axis_index("s") * num_cores + lax.axis_index("c")
        base = wid * rows_per_w

        def fire(g, ib):
            pltpu.sync_copy(idx_hbm.at[pl.ds(base + g * _K, _K)], idx_v[ib])
            for j in range(_K):
                pltpu.async_copy(
                    w_hbm.at[idx_v[ib].at[j]],
                    rows_v[ib].at[pl.ds(j * _LANES, _LANES)],
                    gsem[ib],
                )

        def wait_gathers(ib):
            for j in range(_K):
                pltpu.make_async_copy(
                    w_hbm.at[idx_v[ib].at[j]],
                    rows_v[ib].at[pl.ds(j * _LANES, _LANES)],
                    gsem[ib],
                ).wait()

        def store(g, ib):
            pltpu.async_copy(
                rows_v[ib], out_hbm.at[pl.ds((base + g * _K) * _LANES, gsz)],
                ssem[ib],
            )

        def wait_store(ib):
            pltpu.make_async_copy(
                rows_v[ib], out_hbm.at[pl.ds(base * _LANES, gsz)], ssem[ib]
            ).wait()

        for b in range(_NBUF - 1):
            fire(b, b)

        def body(r, carry):
            for b in range(_NBUF):
                g = r * _NBUF + b
                wait_gathers(b)
                store(g, b)
                bprev = (b - 1) % _NBUF
                gf = g + _NBUF - 1

                @pl.when(gf < groups)
                def _():
                    if b == 0:
                        @pl.when(r > 0)
                        def _():
                            wait_store(bprev)
                    else:
                        wait_store(bprev)
                    fire(gf, bprev)

            return carry

        lax.fori_loop(0, rounds, body, 0)
        for b in range(_NBUF):
            wait_store(b)

    return emb(weight_bf16, idx_rows)


def kernel(token_ids, weight):
    b, h = token_ids.shape
    _, d = weight.shape
    n = b * h
    idx_rows = token_ids.reshape(n // _LANES, _LANES).astype(jnp.int32)
    out = _emb_lookup(weight.astype(jnp.bfloat16), idx_rows)
    return out.reshape(b, h, d).astype(jnp.float32)


# speedup vs baseline: 1.4687x; 1.4687x over previous
import functools

import jax
import jax.numpy as jnp
from jax import lax
from jax.experimental import pallas as pl
from jax.experimental.pallas import tpu as pltpu
from jax.experimental.pallas import tpu_sc as plsc

_LANES = 128
_K = 5
_NBUF = 4


def _emb_lookup(weight, idx_rows):
    R = idx_rows.shape[0]
    _, D = weight.shape
    info = plsc.get_sparse_core_info()
    num_cores = info.num_cores
    nw = num_cores * info.num_subcores
    rows_per_w = R // nw
    groups = rows_per_w // _K
    assert groups % _NBUF == 0
    rounds = groups // _NBUF
    gsz = _K * _LANES

    mesh = plsc.VectorSubcoreMesh(core_axis_name="c", subcore_axis_name="s")

    @functools.partial(
        pl.kernel,
        mesh=mesh,
        compiler_params=pltpu.CompilerParams(use_tc_tiling_on_sc=False),
        out_type=jax.ShapeDtypeStruct((R * _LANES, D), jnp.float32),
        scratch_types=[
            [pltpu.VMEM((_K, _LANES), jnp.int32) for _ in range(_NBUF)],
            [pltpu.VMEM((gsz, D), jnp.float32) for _ in range(_NBUF)],
            [pltpu.SemaphoreType.DMA for _ in range(_NBUF)],
            [pltpu.SemaphoreType.DMA for _ in range(_NBUF)],
        ],
    )
    def emb(w_hbm, idx_hbm, out_hbm, idx_v, rows_v, gsem, ssem):
        wid = lax.axis_index("s") * num_cores + lax.axis_index("c")
        base = wid * rows_per_w

        def fire(g, ib):
            pltpu.sync_copy(idx_hbm.at[pl.ds(base + g * _K, _K)], idx_v[ib])
            for j in range(_K):
                pltpu.async_copy(
                    w_hbm.at[idx_v[ib].at[j]],
                    rows_v[ib].at[pl.ds(j * _LANES, _LANES)],
                    gsem[ib],
                )

        def wait_gathers(ib):
            for j in range(_K):
                pltpu.make_async_copy(
                    w_hbm.at[idx_v[ib].at[j]],
                    rows_v[ib].at[pl.ds(j * _LANES, _LANES)],
                    gsem[ib],
                ).wait()

        def store(g, ib):
            pltpu.async_copy(
                rows_v[ib], out_hbm.at[pl.ds((base + g * _K) * _LANES, gsz)],
                ssem[ib],
            )

        def wait_store(ib):
            pltpu.make_async_copy(
                rows_v[ib], out_hbm.at[pl.ds(base * _LANES, gsz)], ssem[ib]
            ).wait()

        for b in range(_NBUF - 1):
            fire(b, b)

        def body(r, carry):
            for b in range(_NBUF):
                g = r * _NBUF + b
                wait_gathers(b)
                store(g, b)
                bprev = (b - 1) % _NBUF
                gf = g + _NBUF - 1

                @pl.when(gf < groups)
                def _():
                    if b == 0:
                        @pl.when(r > 0)
                        def _():
                            wait_store(bprev)
                    else:
                        wait_store(bprev)
                    fire(gf, bprev)

            return carry

        lax.fori_loop(0, rounds, body, 0)
        for b in range(_NBUF):
            wait_store(b)

    return emb(weight, idx_rows)


def kernel(token_ids, weight):
    b, h = token_ids.shape
    _, d = weight.shape
    n = b * h
    idx_rows = token_ids.reshape(n // _LANES, _LANES).astype(jnp.int32)
    out = _emb_lookup(weight, idx_rows)
    return out.reshape(b, h, d)


# speedup vs baseline: 1.4693x; 1.0004x over previous
import functools

import jax
import jax.numpy as jnp
from jax import lax
from jax.experimental import pallas as pl
from jax.experimental.pallas import tpu as pltpu
from jax.experimental.pallas import tpu_sc as plsc

_KR = 4
_NBUF = 4


def _emb_lookup(weight, token_ids):
    B, H = token_ids.shape
    _, D = weight.shape
    info = plsc.get_sparse_core_info()
    num_cores = info.num_cores
    nw = num_cores * info.num_subcores
    rows_per_w = B // nw
    groups = rows_per_w // _KR
    assert groups % _NBUF == 0
    rounds = groups // _NBUF
    windows = []
    off = 0
    while off < H:
        w = min(128, H - off)
        windows.append((off, w))
        off += w

    mesh = plsc.VectorSubcoreMesh(core_axis_name="c", subcore_axis_name="s")

    @functools.partial(
        pl.kernel,
        mesh=mesh,
        compiler_params=pltpu.CompilerParams(use_tc_tiling_on_sc=False),
        out_type=jax.ShapeDtypeStruct((B, H, D), jnp.float32),
        scratch_types=[
            [pltpu.VMEM((_KR, H), jnp.int32) for _ in range(_NBUF)],
            [pltpu.VMEM((_KR, H, D), jnp.float32) for _ in range(_NBUF)],
            [pltpu.SemaphoreType.DMA for _ in range(_NBUF)],
            [pltpu.SemaphoreType.DMA for _ in range(_NBUF)],
        ],
    )
    def emb(w_hbm, idx_hbm, out_hbm, idx_v, rows_v, gsem, ssem):
        wid = lax.axis_index("s") * num_cores + lax.axis_index("c")
        base = wid * rows_per_w

        def gather_copies(g, ib):
            del g
            return [
                pltpu.make_async_copy(
                    w_hbm.at[idx_v[ib].at[jr, pl.ds(c0, w)]],
                    rows_v[ib].at[jr, pl.ds(c0, w)],
                    gsem[ib],
                )
                for jr in range(_KR)
                for (c0, w) in windows
            ]

        def fire(g, ib):
            pltpu.sync_copy(idx_hbm.at[pl.ds(base + g * _KR, _KR)], idx_v[ib])
            for c in gather_copies(g, ib):
                c.start()

        def wait_gathers(ib):
            for c in gather_copies(0, ib):
                c.wait()

        def store(g, ib):
            pltpu.async_copy(
                rows_v[ib], out_hbm.at[pl.ds(base + g * _KR, _KR)], ssem[ib]
            )

        def wait_store(ib):
            pltpu.make_async_copy(
                rows_v[ib], out_hbm.at[pl.ds(base, _KR)], ssem[ib]
            ).wait()

        for b in range(_NBUF - 1):
            fire(b, b)

        def body(r, carry):
            for b in range(_NBUF):
                g = r * _NBUF + b
                wait_gathers(b)
                store(g, b)
                bprev = (b - 1) % _NBUF
                gf = g + _NBUF - 1

                @pl.when(gf < groups)
                def _():
                    if b == 0:
                        @pl.when(r > 0)
                        def _():
                            wait_store(bprev)
                    else:
                        wait_store(bprev)
                    fire(gf, bprev)

            return carry

        lax.fori_loop(0, rounds, body, 0)
        for b in range(_NBUF):
            wait_store(b)

    return emb(weight, token_ids)


def kernel(token_ids, weight):
    return _emb_lookup(weight, token_ids.astype(jnp.int32))
